# Initial kernel scaffold; baseline (speedup 1.0000x reference)
import jax
import jax.numpy as jnp
from jax.experimental import pallas as pl


def kernel(x, edge_index, W1, b1, W2, b2):
    raise NotImplementedError("write your pallas kernel here")


# speedup vs baseline: 65.3604x; 65.3604x over previous
import functools

import jax
import jax.numpy as jnp
from jax import lax
from jax.experimental import pallas as pl
from jax.experimental.pallas import tpu as pltpu
from jax.experimental.pallas import tpu_sc as plsc

N_NODES = 100000
N_EDGES = 6400000
F = 16
N_PAD = 100352
NC, NS = 2, 16
NW = NC * NS
EW = N_EDGES // NW
C = 80
G = 20
NG = EW // (G * C)
ROWS_PER_TILE = N_PAD // NS
R = 1024
TC_GRID = N_PAD // R


def _worker_id():
    return lax.axis_index("c") * NS + lax.axis_index("s")


def _deg_body(dst3, zdeg, out, deg_local, dst_idx):
    w = _worker_id()
    pltpu.sync_copy(zdeg, deg_local)
    ones16 = jnp.full((16,), 1.0, jnp.float32)

    def group(g, carry):
        pltpu.sync_copy(dst3.at[w * NG + g], dst_idx)
        for k in range(G):
            for j in range(C // 16):
                idx = dst_idx[k, pl.ds(j * 16, 16)]
                plsc.addupdate_scatter(deg_local, [idx], ones16)
        return carry

    lax.fori_loop(0, NG, group, 0)
    pltpu.sync_copy(deg_local, out.at[w])


def _deg_call(dst3, zdeg):
    mesh = plsc.VectorSubcoreMesh(core_axis_name="c", subcore_axis_name="s")
    return pl.kernel(
        _deg_body,
        out_type=jax.ShapeDtypeStruct((NW, N_PAD), jnp.float32),
        mesh=mesh,
        compiler_params=pltpu.CompilerParams(needs_layout_passes=False),
        scratch_types=[
            pltpu.VMEM((N_PAD,), jnp.float32),
            pltpu.VMEM((G, C), jnp.int32),
        ],
    )(dst3, zdeg)


def _agg_body(feat, src3, dst3, zrow, out, acc, src_idx, dst_idx, rows, sem):
    cid = lax.axis_index("c")
    sid = lax.axis_index("s")
    w = cid * NS + sid
    pltpu.sync_copy(zrow, acc.at[pl.ds(sid * ROWS_PER_TILE, ROWS_PER_TILE)])
    plsc.subcore_barrier()

    def group(g, carry):
        pltpu.sync_copy(src3.at[w * NG + g], src_idx)
        pltpu.sync_copy(dst3.at[w * NG + g], dst_idx)
        cps = [
            pltpu.async_copy(feat.at[src_idx.at[k]], rows.at[k], sem)
            for k in range(G)
        ]
        for cp in cps:
            cp.wait()
        for k in range(G):
            pltpu.sync_copy(rows.at[k], acc.at[dst_idx.at[k]], add=True)
        return carry

    lax.fori_loop(0, NG, group, 0)
    plsc.subcore_barrier()
    sl = pl.ds(sid * ROWS_PER_TILE, ROWS_PER_TILE)
    pltpu.sync_copy(acc.at[sl], out.at[cid, sl])


def _agg_call(feat, src3, dst3, zrow):
    mesh = plsc.VectorSubcoreMesh(core_axis_name="c", subcore_axis_name="s")
    return pl.kernel(
        _agg_body,
        out_type=jax.ShapeDtypeStruct((NC, N_PAD, F), jnp.float32),
        mesh=mesh,
        compiler_params=pltpu.CompilerParams(use_tc_tiling_on_sc=False),
        scratch_types=[
            pltpu.VMEM_SHARED((N_PAD, F), jnp.float32),
            pltpu.VMEM((G, C), jnp.int32),
            pltpu.VMEM((G, C), jnp.int32),
            pltpu.VMEM((G, C, F), jnp.float32),
            pltpu.SemaphoreType.DMA,
        ],
    )(feat, src3, dst3, zrow)


def _prescale_body(degp, xb, dis_o, xp_o):
    deg = jnp.sum(degp[...], axis=0) + 1.0
    dis = lax.rsqrt(deg)[:, None]
    dis16 = jnp.broadcast_to(dis, (R, F))
    dis_o[...] = dis16
    xp_o[...] = xb[...] * dis16


def _tc_prescale(deg_parts, x16):
    return pl.pallas_call(
        _prescale_body,
        grid=(TC_GRID,),
        in_specs=[
            pl.BlockSpec((NW, R), lambda i: (0, i)),
            pl.BlockSpec((R, F), lambda i: (i, 0)),
        ],
        out_specs=[
            pl.BlockSpec((R, F), lambda i: (i, 0)),
            pl.BlockSpec((R, F), lambda i: (i, 0)),
        ],
        out_shape=[
            jax.ShapeDtypeStruct((N_PAD, F), jnp.float32),
            jax.ShapeDtypeStruct((N_PAD, F), jnp.float32),
        ],
    )(deg_parts, x16)


def _mid_body(agg, xp, dis, w1, b1, w2, gp_o):
    t = dis[...] * (agg[0] + agg[1] + xp[...])
    h1 = jnp.dot(t, w1[...], preferred_element_type=jnp.float32) + b1[...]
    h1 = jnp.maximum(h1, 0.0)
    g = jnp.dot(h1, w2[...], preferred_element_type=jnp.float32)
    gp_o[...] = dis[...] * g


def _tc_mid(agg1, xp, dis16, w1p, b1r, w2p):
    return pl.pallas_call(
        _mid_body,
        grid=(TC_GRID,),
        in_specs=[
            pl.BlockSpec((NC, R, F), lambda i: (0, i, 0)),
            pl.BlockSpec((R, F), lambda i: (i, 0)),
            pl.BlockSpec((R, F), lambda i: (i, 0)),
            pl.BlockSpec((F, 64), lambda i: (0, 0)),
            pl.BlockSpec((1, 64), lambda i: (0, 0)),
            pl.BlockSpec((64, F), lambda i: (0, 0)),
        ],
        out_specs=pl.BlockSpec((R, F), lambda i: (i, 0)),
        out_shape=jax.ShapeDtypeStruct((N_PAD, F), jnp.float32),
    )(agg1, xp, dis16, w1p, b1r, w2p)


def _final_body(agg, gp, dis, b2m, o):
    h2 = dis[...] * (agg[0] + agg[1] + gp[...]) + b2m[...]
    m = jnp.max(h2, axis=1, keepdims=True)
    e = jnp.exp(h2 - m)
    se = jnp.sum(e, axis=1, keepdims=True)
    o[...] = h2 - m - jnp.log(se)


def _tc_final(agg2, gp, dis16, b2m):
    return pl.pallas_call(
        _final_body,
        grid=(TC_GRID,),
        in_specs=[
            pl.BlockSpec((NC, R, F), lambda i: (0, i, 0)),
            pl.BlockSpec((R, F), lambda i: (i, 0)),
            pl.BlockSpec((R, F), lambda i: (i, 0)),
            pl.BlockSpec((1, F), lambda i: (0, 0)),
        ],
        out_specs=pl.BlockSpec((R, F), lambda i: (i, 0)),
        out_shape=jax.ShapeDtypeStruct((N_PAD, F), jnp.float32),
    )(agg2, gp, dis16, b2m)


def kernel(x, edge_index, W1, b1, W2, b2):
    src = edge_index[0].astype(jnp.int32)
    dst = edge_index[1].astype(jnp.int32)
    src3 = src.reshape(NW * NG, G, C)
    dst3 = dst.reshape(NW * NG, G, C)

    x16 = jnp.pad(x, ((0, N_PAD - N_NODES), (0, F - x.shape[1])))
    w1p = jnp.pad(W1, ((0, F - W1.shape[0]), (0, 0)))
    w2p = jnp.pad(W2, ((0, 0), (0, F - W2.shape[1])))
    b1r = b1.reshape(1, 64)
    b2m = jnp.concatenate([b2, jnp.full((F - b2.shape[0],), -1e30, b2.dtype)])
    b2m = b2m.reshape(1, F)
    zrow = jnp.zeros((ROWS_PER_TILE, F), jnp.float32)
    zdeg = jnp.zeros((N_PAD,), jnp.float32)

    deg_parts = _deg_call(dst3, zdeg)
    dis16, xp = _tc_prescale(deg_parts, x16)
    agg1 = _agg_call(xp, src3, dst3, zrow)
    gp = _tc_mid(agg1, xp, dis16, w1p, b1r, w2p)
    agg2 = _agg_call(gp, src3, dst3, zrow)
    outp = _tc_final(agg2, gp, dis16, b2m)
    return outp[:N_NODES, :10]


# speedup vs baseline: 84.5714x; 1.2939x over previous
import functools

import jax
import jax.numpy as jnp
from jax import lax
from jax.experimental import pallas as pl
from jax.experimental.pallas import tpu as pltpu
from jax.experimental.pallas import tpu_sc as plsc

N_NODES = 100000
N_EDGES = 6400000
F = 16
N_PAD = 100352
N_ACC = 100000
NC, NS = 2, 16
NW = NC * NS
EW = N_EDGES // NW
C = 80
G = 20
NG = EW // (G * C)
GA = 10
NSUP = EW // (GA * C)
ROWS_PER_TILE = N_ACC // NS
R = 1024
TC_GRID = N_PAD // R


def _worker_id():
    return lax.axis_index("c") * NS + lax.axis_index("s")


def _deg_body(dst3, zdeg, out, deg_local, dst_idx):
    w = _worker_id()
    pltpu.sync_copy(zdeg, deg_local)
    ones16 = jnp.full((16,), 1.0, jnp.float32)

    def group(g, carry):
        pltpu.sync_copy(dst3.at[w * NG + g], dst_idx)
        for k in range(G):
            for j in range(C // 16):
                idx = dst_idx[k, pl.ds(j * 16, 16)]
                plsc.addupdate_scatter(deg_local, [idx], ones16)
        return carry

    lax.fori_loop(0, NG, group, 0)
    pltpu.sync_copy(deg_local, out.at[w])


def _deg_call(dst3, zdeg):
    mesh = plsc.VectorSubcoreMesh(core_axis_name="c", subcore_axis_name="s")
    return pl.kernel(
        _deg_body,
        out_type=jax.ShapeDtypeStruct((NW, N_PAD), jnp.float32),
        mesh=mesh,
        compiler_params=pltpu.CompilerParams(needs_layout_passes=False),
        scratch_types=[
            pltpu.VMEM((N_PAD,), jnp.float32),
            pltpu.VMEM((G, C), jnp.int32),
        ],
    )(dst3, zdeg)


def _agg_body(feat, src3, dst3, zrow, out, acc, sidx, didx, rows,
              gs0, gs1, ss0, ss1):
    cid = lax.axis_index("c")
    sid = lax.axis_index("s")
    w = cid * NS + sid
    pltpu.sync_copy(zrow, acc.at[pl.ds(sid * ROWS_PER_TILE, ROWS_PER_TILE)])
    plsc.subcore_barrier()
    gsems = (gs0, gs1)
    ssems = (ss0, ss1)

    def load_and_fire(b, g):
        pltpu.sync_copy(src3.at[w * NSUP + g], sidx.at[b])
        pltpu.sync_copy(dst3.at[w * NSUP + g], didx.at[b])
        for k in range(GA):
            pltpu.async_copy(feat.at[sidx.at[b, k]], rows.at[b, k], gsems[b])

    def wait_gathers(b):
        for k in range(GA):
            pltpu.make_async_copy(
                feat.at[sidx.at[b, k]], rows.at[b, k], gsems[b]).wait()

    def fire_scatters(b):
        for k in range(GA):
            pltpu.async_copy(rows.at[b, k], acc.at[didx.at[b, k]], ssems[b],
                             add=True)

    def wait_scatters(b):
        for k in range(GA):
            pltpu.make_async_copy(
                rows.at[b, k], acc.at[didx.at[b, k]], ssems[b]).wait()

    load_and_fire(0, 0)
    load_and_fire(1, 1)

    def step(i, carry):
        for b in (0, 1):
            wait_gathers(b)
            fire_scatters(b)
            wait_scatters(b)

            @pl.when(i < NSUP // 2 - 1)
            def _():
                load_and_fire(b, 2 * i + b + 2)
        return carry

    lax.fori_loop(0, NSUP // 2, step, 0)
    plsc.subcore_barrier()
    sl = pl.ds(sid * ROWS_PER_TILE, ROWS_PER_TILE)
    pltpu.sync_copy(acc.at[sl], out.at[cid, sl])


def _agg_call(feat, src3, dst3, zrow):
    mesh = plsc.VectorSubcoreMesh(core_axis_name="c", subcore_axis_name="s")
    return pl.kernel(
        _agg_body,
        out_type=jax.ShapeDtypeStruct((NC, N_PAD, F), jnp.float32),
        mesh=mesh,
        compiler_params=pltpu.CompilerParams(use_tc_tiling_on_sc=False),
        scratch_types=[
            pltpu.VMEM_SHARED((N_ACC, F), jnp.float32),
            pltpu.VMEM((2, GA, C), jnp.int32),
            pltpu.VMEM((2, GA, C), jnp.int32),
            pltpu.VMEM((2, GA, C, F), jnp.float32),
            pltpu.SemaphoreType.DMA,
            pltpu.SemaphoreType.DMA,
            pltpu.SemaphoreType.DMA,
            pltpu.SemaphoreType.DMA,
        ],
    )(feat, src3, dst3, zrow)


def _prescale_body(degp, xb, dis_o, xp_o):
    deg = jnp.sum(degp[...], axis=0) + 1.0
    dis = lax.rsqrt(deg)[:, None]
    dis16 = jnp.broadcast_to(dis, (R, F))
    dis_o[...] = dis16
    xp_o[...] = xb[...] * dis16


def _tc_prescale(deg_parts, x16):
    return pl.pallas_call(
        _prescale_body,
        grid=(TC_GRID,),
        in_specs=[
            pl.BlockSpec((NW, R), lambda i: (0, i)),
            pl.BlockSpec((R, F), lambda i: (i, 0)),
        ],
        out_specs=[
            pl.BlockSpec((R, F), lambda i: (i, 0)),
            pl.BlockSpec((R, F), lambda i: (i, 0)),
        ],
        out_shape=[
            jax.ShapeDtypeStruct((N_PAD, F), jnp.float32),
            jax.ShapeDtypeStruct((N_PAD, F), jnp.float32),
        ],
    )(deg_parts, x16)


def _mid_body(agg, xp, dis, w1, b1, w2, gp_o):
    t = dis[...] * (agg[0] + agg[1] + xp[...])
    h1 = jnp.dot(t, w1[...], preferred_element_type=jnp.float32) + b1[...]
    h1 = jnp.maximum(h1, 0.0)
    g = jnp.dot(h1, w2[...], preferred_element_type=jnp.float32)
    gp_o[...] = dis[...] * g


def _tc_mid(agg1, xp, dis16, w1p, b1r, w2p):
    return pl.pallas_call(
        _mid_body,
        grid=(TC_GRID,),
        in_specs=[
            pl.BlockSpec((NC, R, F), lambda i: (0, i, 0)),
            pl.BlockSpec((R, F), lambda i: (i, 0)),
            pl.BlockSpec((R, F), lambda i: (i, 0)),
            pl.BlockSpec((F, 64), lambda i: (0, 0)),
            pl.BlockSpec((1, 64), lambda i: (0, 0)),
            pl.BlockSpec((64, F), lambda i: (0, 0)),
        ],
        out_specs=pl.BlockSpec((R, F), lambda i: (i, 0)),
        out_shape=jax.ShapeDtypeStruct((N_PAD, F), jnp.float32),
    )(agg1, xp, dis16, w1p, b1r, w2p)


def _final_body(agg, gp, dis, b2m, o):
    h2 = dis[...] * (agg[0] + agg[1] + gp[...]) + b2m[...]
    m = jnp.max(h2, axis=1, keepdims=True)
    e = jnp.exp(h2 - m)
    se = jnp.sum(e, axis=1, keepdims=True)
    o[...] = h2 - m - jnp.log(se)


def _tc_final(agg2, gp, dis16, b2m):
    return pl.pallas_call(
        _final_body,
        grid=(TC_GRID,),
        in_specs=[
            pl.BlockSpec((NC, R, F), lambda i: (0, i, 0)),
            pl.BlockSpec((R, F), lambda i: (i, 0)),
            pl.BlockSpec((R, F), lambda i: (i, 0)),
            pl.BlockSpec((1, F), lambda i: (0, 0)),
        ],
        out_specs=pl.BlockSpec((R, F), lambda i: (i, 0)),
        out_shape=jax.ShapeDtypeStruct((N_PAD, F), jnp.float32),
    )(agg2, gp, dis16, b2m)


def kernel(x, edge_index, W1, b1, W2, b2):
    src = edge_index[0].astype(jnp.int32)
    dst = edge_index[1].astype(jnp.int32)
    src3a = src.reshape(NW * NSUP, GA, C)
    dst3a = dst.reshape(NW * NSUP, GA, C)
    dst3d = dst.reshape(NW * NG, G, C)

    x16 = jnp.pad(x, ((0, N_PAD - N_NODES), (0, F - x.shape[1])))
    w1p = jnp.pad(W1, ((0, F - W1.shape[0]), (0, 0)))
    w2p = jnp.pad(W2, ((0, 0), (0, F - W2.shape[1])))
    b1r = b1.reshape(1, 64)
    b2m = jnp.concatenate([b2, jnp.full((F - b2.shape[0],), -1e30, b2.dtype)])
    b2m = b2m.reshape(1, F)
    zrow = jnp.zeros((ROWS_PER_TILE, F), jnp.float32)
    zdeg = jnp.zeros((N_PAD,), jnp.float32)

    deg_parts = _deg_call(dst3d, zdeg)
    dis16, xp = _tc_prescale(deg_parts, x16)
    agg1 = _agg_call(xp, src3a, dst3a, zrow)
    gp = _tc_mid(agg1, xp, dis16, w1p, b1r, w2p)
    agg2 = _agg_call(gp, src3a, dst3a, zrow)
    outp = _tc_final(agg2, gp, dis16, b2m)
    return outp[:N_NODES, :10]


# speedup vs baseline: 100.2056x; 1.1849x over previous
import functools

import jax
import jax.numpy as jnp
from jax import lax
from jax.experimental import pallas as pl
from jax.experimental.pallas import tpu as pltpu
from jax.experimental.pallas import tpu_sc as plsc

N_NODES = 100000
N_EDGES = 6400000
F = 16
N_PAD = 100352
N_ACC = 100000
NC, NS = 2, 16
NW = NC * NS
EW = N_EDGES // NW
GC = 1600
NG = EW // GC
SUP = 800
NSUP = EW // SUP
ROWS_PER_TILE = N_ACC // NS
R = 1024
TC_GRID = N_PAD // R


def _worker_id():
    return lax.axis_index("c") * NS + lax.axis_index("s")


def _deg_body(dst1, zdeg, out, deg_local, dst_idx):
    w = _worker_id()
    pltpu.sync_copy(zdeg, deg_local)
    ones16 = jnp.full((16,), 1.0, jnp.float32)

    def group(g, carry):
        pltpu.sync_copy(dst1.at[pl.ds((w * NG + g) * GC, GC)], dst_idx)
        for j in range(GC // 16):
            idx = dst_idx[pl.ds(j * 16, 16)]
            plsc.addupdate_scatter(deg_local, [idx], ones16)
        return carry

    lax.fori_loop(0, NG, group, 0)
    pltpu.sync_copy(deg_local, out.at[w])


def _deg_call(dst1, zdeg):
    mesh = plsc.VectorSubcoreMesh(core_axis_name="c", subcore_axis_name="s")
    return pl.kernel(
        _deg_body,
        out_type=jax.ShapeDtypeStruct((NW, N_PAD), jnp.float32),
        mesh=mesh,
        compiler_params=pltpu.CompilerParams(needs_layout_passes=False),
        scratch_types=[
            pltpu.VMEM((N_PAD,), jnp.float32),
            pltpu.VMEM((GC,), jnp.int32),
        ],
    )(dst1, zdeg)


def _agg_body(feat, src1, dst1, zrow, out, acc, sidx, didx, rows,
              gs0, gs1, ss0, ss1):
    cid = lax.axis_index("c")
    sid = lax.axis_index("s")
    w = cid * NS + sid
    pltpu.sync_copy(zrow, acc.at[pl.ds(sid * ROWS_PER_TILE, ROWS_PER_TILE)])
    plsc.subcore_barrier()
    gsems = (gs0, gs1)
    ssems = (ss0, ss1)

    def load_and_fire(b, g):
        base = (w * NSUP + g) * SUP
        pltpu.sync_copy(src1.at[pl.ds(base, SUP)], sidx.at[b])
        pltpu.sync_copy(dst1.at[pl.ds(base, SUP)], didx.at[b])
        pltpu.async_copy(feat.at[sidx.at[b]], rows.at[b], gsems[b])

    def step(i, carry):
        for b in (0, 1):
            pltpu.make_async_copy(feat.at[sidx.at[b]], rows.at[b],
                                  gsems[b]).wait()
            pltpu.async_copy(rows.at[b], acc.at[didx.at[b]], ssems[b],
                             add=True)
            pltpu.make_async_copy(rows.at[b], acc.at[didx.at[b]],
                                  ssems[b]).wait()

            @pl.when(i < NSUP // 2 - 1)
            def _():
                load_and_fire(b, 2 * i + b + 2)
        return carry

    load_and_fire(0, 0)
    load_and_fire(1, 1)
    lax.fori_loop(0, NSUP // 2, step, 0)
    plsc.subcore_barrier()
    sl = pl.ds(sid * ROWS_PER_TILE, ROWS_PER_TILE)
    pltpu.sync_copy(acc.at[sl], out.at[cid, sl])


def _agg_call(feat, src1, dst1, zrow):
    mesh = plsc.VectorSubcoreMesh(core_axis_name="c", subcore_axis_name="s")
    return pl.kernel(
        _agg_body,
        out_type=jax.ShapeDtypeStruct((NC, N_PAD, F), jnp.float32),
        mesh=mesh,
        compiler_params=pltpu.CompilerParams(use_tc_tiling_on_sc=False),
        scratch_types=[
            pltpu.VMEM_SHARED((N_ACC, F), jnp.float32),
            pltpu.VMEM((2, SUP), jnp.int32),
            pltpu.VMEM((2, SUP), jnp.int32),
            pltpu.VMEM((2, SUP, F), jnp.float32),
            pltpu.SemaphoreType.DMA,
            pltpu.SemaphoreType.DMA,
            pltpu.SemaphoreType.DMA,
            pltpu.SemaphoreType.DMA,
        ],
    )(feat, src1, dst1, zrow)


def _prescale_body(degp, xb, dis_o, xp_o):
    deg = jnp.sum(degp[...], axis=0) + 1.0
    dis = lax.rsqrt(deg)[:, None]
    dis16 = jnp.broadcast_to(dis, (R, F))
    dis_o[...] = dis16
    xp_o[...] = xb[...] * dis16


def _tc_prescale(deg_parts, x16):
    return pl.pallas_call(
        _prescale_body,
        grid=(TC_GRID,),
        in_specs=[
            pl.BlockSpec((NW, R), lambda i: (0, i)),
            pl.BlockSpec((R, F), lambda i: (i, 0)),
        ],
        out_specs=[
            pl.BlockSpec((R, F), lambda i: (i, 0)),
            pl.BlockSpec((R, F), lambda i: (i, 0)),
        ],
        out_shape=[
            jax.ShapeDtypeStruct((N_PAD, F), jnp.float32),
            jax.ShapeDtypeStruct((N_PAD, F), jnp.float32),
        ],
    )(deg_parts, x16)


def _mid_body(agg, xp, dis, w1, b1, w2, gp_o):
    t = dis[...] * (agg[0] + agg[1] + xp[...])
    h1 = jnp.dot(t, w1[...], preferred_element_type=jnp.float32) + b1[...]
    h1 = jnp.maximum(h1, 0.0)
    g = jnp.dot(h1, w2[...], preferred_element_type=jnp.float32)
    gp_o[...] = dis[...] * g


def _tc_mid(agg1, xp, dis16, w1p, b1r, w2p):
    return pl.pallas_call(
        _mid_body,
        grid=(TC_GRID,),
        in_specs=[
            pl.BlockSpec((NC, R, F), lambda i: (0, i, 0)),
            pl.BlockSpec((R, F), lambda i: (i, 0)),
            pl.BlockSpec((R, F), lambda i: (i, 0)),
            pl.BlockSpec((F, 64), lambda i: (0, 0)),
            pl.BlockSpec((1, 64), lambda i: (0, 0)),
            pl.BlockSpec((64, F), lambda i: (0, 0)),
        ],
        out_specs=pl.BlockSpec((R, F), lambda i: (i, 0)),
        out_shape=jax.ShapeDtypeStruct((N_PAD, F), jnp.float32),
    )(agg1, xp, dis16, w1p, b1r, w2p)


def _final_body(agg, gp, dis, b2m, o):
    h2 = dis[...] * (agg[0] + agg[1] + gp[...]) + b2m[...]
    m = jnp.max(h2, axis=1, keepdims=True)
    e = jnp.exp(h2 - m)
    se = jnp.sum(e, axis=1, keepdims=True)
    o[...] = h2 - m - jnp.log(se)


def _tc_final(agg2, gp, dis16, b2m):
    return pl.pallas_call(
        _final_body,
        grid=(TC_GRID,),
        in_specs=[
            pl.BlockSpec((NC, R, F), lambda i: (0, i, 0)),
            pl.BlockSpec((R, F), lambda i: (i, 0)),
            pl.BlockSpec((R, F), lambda i: (i, 0)),
            pl.BlockSpec((1, F), lambda i: (0, 0)),
        ],
        out_specs=pl.BlockSpec((R, F), lambda i: (i, 0)),
        out_shape=jax.ShapeDtypeStruct((N_PAD, F), jnp.float32),
    )(agg2, gp, dis16, b2m)


def kernel(x, edge_index, W1, b1, W2, b2):
    src = edge_index[0].astype(jnp.int32)
    dst = edge_index[1].astype(jnp.int32)

    x16 = jnp.pad(x, ((0, N_PAD - N_NODES), (0, F - x.shape[1])))
    w1p = jnp.pad(W1, ((0, F - W1.shape[0]), (0, 0)))
    w2p = jnp.pad(W2, ((0, 0), (0, F - W2.shape[1])))
    b1r = b1.reshape(1, 64)
    b2m = jnp.concatenate([b2, jnp.full((F - b2.shape[0],), -1e30, b2.dtype)])
    b2m = b2m.reshape(1, F)
    zrow = jnp.zeros((ROWS_PER_TILE, F), jnp.float32)
    zdeg = jnp.zeros((N_PAD,), jnp.float32)

    deg_parts = _deg_call(dst, zdeg)
    dis16, xp = _tc_prescale(deg_parts, x16)
    agg1 = _agg_call(xp, src, dst, zrow)
    gp = _tc_mid(agg1, xp, dis16, w1p, b1r, w2p)
    agg2 = _agg_call(gp, src, dst, zrow)
    outp = _tc_final(agg2, gp, dis16, b2m)
    return outp[:N_NODES, :10]


# speedup vs baseline: 106.6577x; 1.0644x over previous
import functools

import jax
import jax.numpy as jnp
from jax import lax
from jax.experimental import pallas as pl
from jax.experimental.pallas import tpu as pltpu
from jax.experimental.pallas import tpu_sc as plsc

N_NODES = 100000
N_EDGES = 6400000
F = 16
N_PAD = 100352
N_ACC = 100000
NC, NS = 2, 16
NW = NC * NS
EW = N_EDGES // NW
GC = 1600
NG = EW // GC
SUP = 800
NSUP = EW // SUP
ROWS_PER_TILE = N_ACC // NS
R = 7168
TC_GRID = N_PAD // R


def _worker_id():
    return lax.axis_index("c") * NS + lax.axis_index("s")


def _deg_body(dst1, zdeg, out, deg_local, dst_idx):
    w = _worker_id()
    pltpu.sync_copy(zdeg, deg_local)
    ones16 = jnp.full((16,), 1.0, jnp.float32)

    def group(g, carry):
        pltpu.sync_copy(dst1.at[pl.ds((w * NG + g) * GC, GC)], dst_idx)
        for j in range(GC // 16):
            idx = dst_idx[pl.ds(j * 16, 16)]
            plsc.addupdate_scatter(deg_local, [idx], ones16)
        return carry

    lax.fori_loop(0, NG, group, 0)
    pltpu.sync_copy(deg_local, out.at[pl.ds(w * N_PAD, N_PAD)])


def _deg_call(dst1, zdeg):
    mesh = plsc.VectorSubcoreMesh(core_axis_name="c", subcore_axis_name="s")
    return pl.kernel(
        _deg_body,
        out_type=jax.ShapeDtypeStruct((NW * N_PAD,), jnp.float32),
        mesh=mesh,
        compiler_params=pltpu.CompilerParams(needs_layout_passes=False),
        scratch_types=[
            pltpu.VMEM((N_PAD,), jnp.float32),
            pltpu.VMEM((GC,), jnp.int32),
        ],
    )(dst1, zdeg)


def _agg_body(feat, src1, dst1, zrow, out, acc, sidx, didx, rows,
              gs0, gs1, ss0, ss1):
    cid = lax.axis_index("c")
    sid = lax.axis_index("s")
    w = cid * NS + sid
    pltpu.sync_copy(zrow, acc.at[pl.ds(sid * ROWS_PER_TILE, ROWS_PER_TILE)])
    plsc.subcore_barrier()
    gsems = (gs0, gs1)
    ssems = (ss0, ss1)

    def load_and_fire(b, g):
        base = (w * NSUP + g) * SUP
        pltpu.sync_copy(src1.at[pl.ds(base, SUP)], sidx.at[b])
        pltpu.sync_copy(dst1.at[pl.ds(base, SUP)], didx.at[b])
        pltpu.async_copy(feat.at[sidx.at[b]], rows.at[b], gsems[b])

    def step(i, carry):
        for b in (0, 1):
            pltpu.make_async_copy(feat.at[sidx.at[b]], rows.at[b],
                                  gsems[b]).wait()
            pltpu.async_copy(rows.at[b], acc.at[didx.at[b]], ssems[b],
                             add=True)
            pltpu.make_async_copy(rows.at[b], acc.at[didx.at[b]],
                                  ssems[b]).wait()

            @pl.when(i < NSUP // 2 - 1)
            def _():
                load_and_fire(b, 2 * i + b + 2)
        return carry

    load_and_fire(0, 0)
    load_and_fire(1, 1)
    lax.fori_loop(0, NSUP // 2, step, 0)
    plsc.subcore_barrier()
    sl = pl.ds(sid * ROWS_PER_TILE, ROWS_PER_TILE)
    pltpu.sync_copy(acc.at[sl], out.at[cid, sl])


def _agg_call(feat, src1, dst1, zrow):
    mesh = plsc.VectorSubcoreMesh(core_axis_name="c", subcore_axis_name="s")
    return pl.kernel(
        _agg_body,
        out_type=jax.ShapeDtypeStruct((NC, N_PAD, F), jnp.float32),
        mesh=mesh,
        compiler_params=pltpu.CompilerParams(use_tc_tiling_on_sc=False),
        scratch_types=[
            pltpu.VMEM_SHARED((N_ACC, F), jnp.float32),
            pltpu.VMEM((2, SUP), jnp.int32),
            pltpu.VMEM((2, SUP), jnp.int32),
            pltpu.VMEM((2, SUP, F), jnp.float32),
            pltpu.SemaphoreType.DMA,
            pltpu.SemaphoreType.DMA,
            pltpu.SemaphoreType.DMA,
            pltpu.SemaphoreType.DMA,
        ],
    )(feat, src1, dst1, zrow)


def _prescale_body(degp, xb, dis_o, xp_o):
    deg = jnp.sum(degp[...], axis=0) + 1.0
    dis = lax.rsqrt(deg)
    eye = jnp.eye(128, dtype=jnp.float32)
    ones = jnp.ones((128, F), jnp.float32)
    pieces = []
    for q in range(R // 128):
        d = jnp.broadcast_to(dis[q:q + 1, :], (128, 128)) * eye
        pieces.append(jnp.dot(d, ones, preferred_element_type=jnp.float32))
    dis16 = jnp.concatenate(pieces, axis=0)
    dis_o[...] = dis16
    xp_o[...] = xb[...] * dis16


def _tc_prescale(deg_parts, x16):
    return pl.pallas_call(
        _prescale_body,
        grid=(TC_GRID,),
        in_specs=[
            pl.BlockSpec((NW, R // 128, 128), lambda i: (0, i, 0)),
            pl.BlockSpec((R, F), lambda i: (i, 0)),
        ],
        out_specs=[
            pl.BlockSpec((R, F), lambda i: (i, 0)),
            pl.BlockSpec((R, F), lambda i: (i, 0)),
        ],
        out_shape=[
            jax.ShapeDtypeStruct((N_PAD, F), jnp.float32),
            jax.ShapeDtypeStruct((N_PAD, F), jnp.float32),
        ],
    )(deg_parts, x16)


def _mid_body(agg, xp, dis, w1, b1, w2, gp_o):
    t = dis[...] * (agg[0] + agg[1] + xp[...])
    h1 = jnp.dot(t, w1[...], preferred_element_type=jnp.float32) + b1[...]
    h1 = jnp.maximum(h1, 0.0)
    g = jnp.dot(h1, w2[...], preferred_element_type=jnp.float32)
    gp_o[...] = dis[...] * g


def _tc_mid(agg1, xp, dis16, w1p, b1r, w2p):
    return pl.pallas_call(
        _mid_body,
        grid=(TC_GRID,),
        in_specs=[
            pl.BlockSpec((NC, R, F), lambda i: (0, i, 0)),
            pl.BlockSpec((R, F), lambda i: (i, 0)),
            pl.BlockSpec((R, F), lambda i: (i, 0)),
            pl.BlockSpec((F, 64), lambda i: (0, 0)),
            pl.BlockSpec((1, 64), lambda i: (0, 0)),
            pl.BlockSpec((64, F), lambda i: (0, 0)),
        ],
        out_specs=pl.BlockSpec((R, F), lambda i: (i, 0)),
        out_shape=jax.ShapeDtypeStruct((N_PAD, F), jnp.float32),
    )(agg1, xp, dis16, w1p, b1r, w2p)


def _final_body(agg, gp, dis, b2m, o):
    h2 = dis[...] * (agg[0] + agg[1] + gp[...]) + b2m[...]
    m = jnp.max(h2, axis=1, keepdims=True)
    e = jnp.exp(h2 - m)
    se = jnp.sum(e, axis=1, keepdims=True)
    o[...] = h2 - m - jnp.log(se)


def _tc_final(agg2, gp, dis16, b2m):
    return pl.pallas_call(
        _final_body,
        grid=(TC_GRID,),
        in_specs=[
            pl.BlockSpec((NC, R, F), lambda i: (0, i, 0)),
            pl.BlockSpec((R, F), lambda i: (i, 0)),
            pl.BlockSpec((R, F), lambda i: (i, 0)),
            pl.BlockSpec((1, F), lambda i: (0, 0)),
        ],
        out_specs=pl.BlockSpec((R, F), lambda i: (i, 0)),
        out_shape=jax.ShapeDtypeStruct((N_PAD, F), jnp.float32),
    )(agg2, gp, dis16, b2m)


def kernel(x, edge_index, W1, b1, W2, b2):
    src = edge_index[0].astype(jnp.int32)
    dst = edge_index[1].astype(jnp.int32)

    x16 = jnp.pad(x, ((0, N_PAD - N_NODES), (0, F - x.shape[1])))
    w1p = jnp.pad(W1, ((0, F - W1.shape[0]), (0, 0)))
    w2p = jnp.pad(W2, ((0, 0), (0, F - W2.shape[1])))
    b1r = b1.reshape(1, 64)
    b2m = jnp.concatenate([b2, jnp.full((F - b2.shape[0],), -1e30, b2.dtype)])
    b2m = b2m.reshape(1, F)
    zrow = jnp.zeros((ROWS_PER_TILE, F), jnp.float32)
    zdeg = jnp.zeros((N_PAD,), jnp.float32)

    deg_flat = _deg_call(dst, zdeg)
    deg_parts = deg_flat.reshape(NW, N_PAD // 128, 128)
    dis16, xp = _tc_prescale(deg_parts, x16)
    agg1 = _agg_call(xp, src, dst, zrow)
    gp = _tc_mid(agg1, xp, dis16, w1p, b1r, w2p)
    agg2 = _agg_call(gp, src, dst, zrow)
    outp = _tc_final(agg2, gp, dis16, b2m)
    return outp[:N_NODES, :10]


# speedup vs baseline: 129.1149x; 1.2106x over previous
import functools

import jax
import jax.numpy as jnp
from jax import lax
from jax.experimental import pallas as pl
from jax.experimental.pallas import tpu as pltpu
from jax.experimental.pallas import tpu_sc as plsc

N_NODES = 100000
N_EDGES = 6400000
F = 16
N_PAD = 100352
N_ACC = 100000
NC, NS = 2, 16
NW = NC * NS
EW = N_EDGES // NW
GC = 1600
NG = EW // GC
SUP = 800
NSUP = EW // SUP
ROWS_PER_TILE = N_ACC // NS
R = 7168
TC_GRID = N_PAD // R


def _worker_id():
    return lax.axis_index("c") * NS + lax.axis_index("s")


def _deg_body(dst1, zdeg, out, deg_local, dst_idx):
    w = _worker_id()
    pltpu.sync_copy(zdeg, deg_local)
    ones16 = jnp.full((16,), 1.0, jnp.float32)

    def group(g, carry):
        pltpu.sync_copy(dst1.at[pl.ds((w * NG + g) * GC, GC)], dst_idx)
        for j in range(GC // 16):
            idx = dst_idx[pl.ds(j * 16, 16)]
            plsc.addupdate_scatter(deg_local, [idx], ones16)
        return carry

    lax.fori_loop(0, NG, group, 0)
    pltpu.sync_copy(deg_local, out.at[pl.ds(w * N_PAD, N_PAD)])


def _deg_call(dst1, zdeg):
    mesh = plsc.VectorSubcoreMesh(core_axis_name="c", subcore_axis_name="s")
    return pl.kernel(
        _deg_body,
        out_type=jax.ShapeDtypeStruct((NW * N_PAD,), jnp.float32),
        mesh=mesh,
        compiler_params=pltpu.CompilerParams(needs_layout_passes=False),
        scratch_types=[
            pltpu.VMEM((N_PAD,), jnp.float32),
            pltpu.VMEM((GC,), jnp.int32),
        ],
    )(dst1, zdeg)


def _agg_body(feat, src1, dst1, zrow, out, acc, sidx, didx, rows,
              gs0, gs1, ss0, ss1, is0, is1, id0, id1):
    cid = lax.axis_index("c")
    sid = lax.axis_index("s")
    w = cid * NS + sid
    pltpu.sync_copy(zrow, acc.at[pl.ds(sid * ROWS_PER_TILE, ROWS_PER_TILE)])
    plsc.subcore_barrier()
    gsems = (gs0, gs1)
    ssems = (ss0, ss1)
    isems = (is0, is1)
    idsems = (id0, id1)

    def base_of(g):
        return (w * NSUP + g) * SUP

    for b in (0, 1):
        pltpu.sync_copy(src1.at[pl.ds(base_of(b), SUP)], sidx.at[b])
        pltpu.async_copy(dst1.at[pl.ds(base_of(b), SUP)], didx.at[b],
                         idsems[b])
        pltpu.async_copy(feat.at[sidx.at[b]], rows.at[b], gsems[b])

    def step(i, carry):
        for b in (0, 1):
            g2 = 2 * i + b + 2
            guard = i < NSUP // 2 - 1
            pltpu.make_async_copy(feat.at[sidx.at[b]], rows.at[b],
                                  gsems[b]).wait()

            @pl.when(guard)
            def _():
                pltpu.async_copy(src1.at[pl.ds(base_of(g2), SUP)],
                                 sidx.at[b], isems[b])

            pltpu.make_async_copy(dst1.at[pl.ds(base_of(g2), SUP)],
                                  didx.at[b], idsems[b]).wait()
            pltpu.async_copy(rows.at[b], acc.at[didx.at[b]], ssems[b],
                             add=True)
            pltpu.make_async_copy(rows.at[b], acc.at[didx.at[b]],
                                  ssems[b]).wait()

            @pl.when(guard)
            def _():
                pltpu.async_copy(dst1.at[pl.ds(base_of(g2), SUP)],
                                 didx.at[b], idsems[b])
                pltpu.make_async_copy(src1.at[pl.ds(base_of(g2), SUP)],
                                      sidx.at[b], isems[b]).wait()
                pltpu.async_copy(feat.at[sidx.at[b]], rows.at[b], gsems[b])
        return carry

    lax.fori_loop(0, NSUP // 2, step, 0)
    plsc.subcore_barrier()
    sl = pl.ds(sid * ROWS_PER_TILE, ROWS_PER_TILE)
    pltpu.sync_copy(acc.at[sl], out.at[cid, sl])


def _agg_call(feat, src1, dst1, zrow):
    mesh = plsc.VectorSubcoreMesh(core_axis_name="c", subcore_axis_name="s")
    return pl.kernel(
        _agg_body,
        out_type=jax.ShapeDtypeStruct((NC, N_PAD, F), jnp.float32),
        mesh=mesh,
        compiler_params=pltpu.CompilerParams(use_tc_tiling_on_sc=False),
        scratch_types=[
            pltpu.VMEM_SHARED((N_ACC, F), jnp.float32),
            pltpu.VMEM((2, SUP), jnp.int32),
            pltpu.VMEM((2, SUP), jnp.int32),
            pltpu.VMEM((2, SUP, F), jnp.float32),
            pltpu.SemaphoreType.DMA,
            pltpu.SemaphoreType.DMA,
            pltpu.SemaphoreType.DMA,
            pltpu.SemaphoreType.DMA,
            pltpu.SemaphoreType.DMA,
            pltpu.SemaphoreType.DMA,
            pltpu.SemaphoreType.DMA,
            pltpu.SemaphoreType.DMA,
        ],
    )(feat, src1, dst1, zrow)


def _prescale_body(degp, xb, dis_o, xp_o):
    deg = jnp.sum(degp[...], axis=0) + 1.0
    dis = lax.rsqrt(deg)
    eye = jnp.eye(128, dtype=jnp.float32)
    ones = jnp.ones((128, F), jnp.float32)
    pieces = []
    for q in range(R // 128):
        d = jnp.broadcast_to(dis[q:q + 1, :], (128, 128)) * eye
        pieces.append(jnp.dot(d, ones, preferred_element_type=jnp.float32))
    dis16 = jnp.concatenate(pieces, axis=0)
    dis_o[...] = dis16
    xp_o[...] = xb[...] * dis16


def _tc_prescale(deg_parts, x16):
    return pl.pallas_call(
        _prescale_body,
        grid=(TC_GRID,),
        in_specs=[
            pl.BlockSpec((NW, R // 128, 128), lambda i: (0, i, 0)),
            pl.BlockSpec((R, F), lambda i: (i, 0)),
        ],
        out_specs=[
            pl.BlockSpec((R, F), lambda i: (i, 0)),
            pl.BlockSpec((R, F), lambda i: (i, 0)),
        ],
        out_shape=[
            jax.ShapeDtypeStruct((N_PAD, F), jnp.float32),
            jax.ShapeDtypeStruct((N_PAD, F), jnp.float32),
        ],
    )(deg_parts, x16)


def _mid_body(agg, xp, dis, w1, b1, w2, gp_o):
    t = dis[...] * (agg[0] + agg[1] + xp[...])
    h1 = jnp.dot(t, w1[...], preferred_element_type=jnp.float32) + b1[...]
    h1 = jnp.maximum(h1, 0.0)
    g = jnp.dot(h1, w2[...], preferred_element_type=jnp.float32)
    gp_o[...] = dis[...] * g


def _tc_mid(agg1, xp, dis16, w1p, b1r, w2p):
    return pl.pallas_call(
        _mid_body,
        grid=(TC_GRID,),
        in_specs=[
            pl.BlockSpec((NC, R, F), lambda i: (0, i, 0)),
            pl.BlockSpec((R, F), lambda i: (i, 0)),
            pl.BlockSpec((R, F), lambda i: (i, 0)),
            pl.BlockSpec((F, 64), lambda i: (0, 0)),
            pl.BlockSpec((1, 64), lambda i: (0, 0)),
            pl.BlockSpec((64, F), lambda i: (0, 0)),
        ],
        out_specs=pl.BlockSpec((R, F), lambda i: (i, 0)),
        out_shape=jax.ShapeDtypeStruct((N_PAD, F), jnp.float32),
    )(agg1, xp, dis16, w1p, b1r, w2p)


def _final_body(agg, gp, dis, b2m, o):
    h2 = dis[...] * (agg[0] + agg[1] + gp[...]) + b2m[...]
    m = jnp.max(h2, axis=1, keepdims=True)
    e = jnp.exp(h2 - m)
    se = jnp.sum(e, axis=1, keepdims=True)
    o[...] = h2 - m - jnp.log(se)


def _tc_final(agg2, gp, dis16, b2m):
    return pl.pallas_call(
        _final_body,
        grid=(TC_GRID,),
        in_specs=[
            pl.BlockSpec((NC, R, F), lambda i: (0, i, 0)),
            pl.BlockSpec((R, F), lambda i: (i, 0)),
            pl.BlockSpec((R, F), lambda i: (i, 0)),
            pl.BlockSpec((1, F), lambda i: (0, 0)),
        ],
        out_specs=pl.BlockSpec((R, F), lambda i: (i, 0)),
        out_shape=jax.ShapeDtypeStruct((N_PAD, F), jnp.float32),
    )(agg2, gp, dis16, b2m)


def kernel(x, edge_index, W1, b1, W2, b2):
    src = edge_index[0].astype(jnp.int32)
    dst = edge_index[1].astype(jnp.int32)

    x16 = jnp.pad(x, ((0, N_PAD - N_NODES), (0, F - x.shape[1])))
    w1p = jnp.pad(W1, ((0, F - W1.shape[0]), (0, 0)))
    w2p = jnp.pad(W2, ((0, 0), (0, F - W2.shape[1])))
    b1r = b1.reshape(1, 64)
    b2m = jnp.concatenate([b2, jnp.full((F - b2.shape[0],), -1e30, b2.dtype)])
    b2m = b2m.reshape(1, F)
    zrow = jnp.zeros((ROWS_PER_TILE, F), jnp.float32)
    zdeg = jnp.zeros((N_PAD,), jnp.float32)

    deg_flat = _deg_call(dst, zdeg)
    deg_parts = deg_flat.reshape(NW, N_PAD // 128, 128)
    dis16, xp = _tc_prescale(deg_parts, x16)
    agg1 = _agg_call(xp, src, dst, zrow)
    gp = _tc_mid(agg1, xp, dis16, w1p, b1r, w2p)
    agg2 = _agg_call(gp, src, dst, zrow)
    outp = _tc_final(agg2, gp, dis16, b2m)
    return outp[:N_NODES, :10]


# speedup vs baseline: 140.7303x; 1.0900x over previous
import functools

import jax
import jax.numpy as jnp
from jax import lax
from jax.experimental import pallas as pl
from jax.experimental.pallas import tpu as pltpu
from jax.experimental.pallas import tpu_sc as plsc

N_NODES = 100000
N_EDGES = 6400000
F = 16
N_PAD = 100352
N_ACC = 100000
NC, NS = 2, 16
NW = NC * NS
EW = N_EDGES // NW
GC = 2000
NG = EW // GC
SUP = 800
NSUP = EW // SUP
ROWS_PER_TILE = N_ACC // NS
R = 7168
RF = 5000
TC_GRID = N_PAD // R


def _worker_id():
    return lax.axis_index("c") * NS + lax.axis_index("s")


def _deg_body(dst1, zdeg, out, deg_local, dst_idx0, dst_idx1, ds0, ds1):
    w = _worker_id()
    pltpu.sync_copy(zdeg, deg_local)
    ones16 = jnp.full((16,), 1.0, jnp.float32)
    bufs = (dst_idx0, dst_idx1)
    dsems = (ds0, ds1)

    def fire(b, g):
        pltpu.async_copy(dst1.at[pl.ds((w * NG + g) * GC, GC)], bufs[b],
                         dsems[b])

    fire(0, 0)
    fire(1, 1)

    def pair(i, carry):
        for b in (0, 1):
            pltpu.make_async_copy(dst1.at[pl.ds(w * GC, GC)], bufs[b],
                                  dsems[b]).wait()
            for j in range(GC // 16):
                idx = bufs[b][pl.ds(j * 16, 16)]
                plsc.addupdate_scatter(deg_local, [idx], ones16)

            @pl.when(i < NG // 2 - 1)
            def _():
                fire(b, 2 * i + b + 2)
        return carry

    lax.fori_loop(0, NG // 2, pair, 0)
    pltpu.sync_copy(deg_local, out.at[pl.ds(w * N_PAD, N_PAD)])


def _deg_call(dst1, zdeg):
    mesh = plsc.VectorSubcoreMesh(core_axis_name="c", subcore_axis_name="s")
    return pl.kernel(
        _deg_body,
        out_type=jax.ShapeDtypeStruct((NW * N_PAD,), jnp.float32),
        mesh=mesh,
        compiler_params=pltpu.CompilerParams(needs_layout_passes=False),
        scratch_types=[
            pltpu.VMEM((N_PAD,), jnp.float32),
            pltpu.VMEM((GC,), jnp.int32),
            pltpu.VMEM((GC,), jnp.int32),
            pltpu.SemaphoreType.DMA,
            pltpu.SemaphoreType.DMA,
        ],
    )(dst1, zdeg)


def _agg_body(feat, src1, dst1, zrow, out, acc, sidx, didx, rows,
              gs0, gs1, ss0, ss1, is0, is1, id0, id1):
    cid = lax.axis_index("c")
    sid = lax.axis_index("s")
    w = cid * NS + sid
    pltpu.sync_copy(zrow, acc.at[pl.ds(sid * ROWS_PER_TILE, ROWS_PER_TILE)])
    plsc.subcore_barrier()
    gsems = (gs0, gs1)
    ssems = (ss0, ss1)
    isems = (is0, is1)
    idsems = (id0, id1)

    def base_of(g):
        return (w * NSUP + g) * SUP

    for b in (0, 1):
        pltpu.sync_copy(src1.at[pl.ds(base_of(b), SUP)], sidx.at[b])
        pltpu.async_copy(dst1.at[pl.ds(base_of(b), SUP)], didx.at[b],
                         idsems[b])
        pltpu.async_copy(feat.at[sidx.at[b]], rows.at[b], gsems[b])

    def step(i, carry):
        for b in (0, 1):
            g2 = 2 * i + b + 2
            guard = i < NSUP // 2 - 1
            pltpu.make_async_copy(feat.at[sidx.at[b]], rows.at[b],
                                  gsems[b]).wait()

            @pl.when(guard)
            def _():
                pltpu.async_copy(src1.at[pl.ds(base_of(g2), SUP)],
                                 sidx.at[b], isems[b])

            pltpu.make_async_copy(dst1.at[pl.ds(base_of(g2), SUP)],
                                  didx.at[b], idsems[b]).wait()
            pltpu.async_copy(rows.at[b], acc.at[didx.at[b]], ssems[b],
                             add=True)
            pltpu.make_async_copy(rows.at[b], acc.at[didx.at[b]],
                                  ssems[b]).wait()

            @pl.when(guard)
            def _():
                pltpu.async_copy(dst1.at[pl.ds(base_of(g2), SUP)],
                                 didx.at[b], idsems[b])
                pltpu.make_async_copy(src1.at[pl.ds(base_of(g2), SUP)],
                                      sidx.at[b], isems[b]).wait()
                pltpu.async_copy(feat.at[sidx.at[b]], rows.at[b], gsems[b])
        return carry

    lax.fori_loop(0, NSUP // 2, step, 0)
    plsc.subcore_barrier()
    sl = pl.ds(sid * ROWS_PER_TILE, ROWS_PER_TILE)
    pltpu.sync_copy(acc.at[sl], out.at[cid, sl])


def _agg_call(feat, src1, dst1, zrow):
    mesh = plsc.VectorSubcoreMesh(core_axis_name="c", subcore_axis_name="s")
    return pl.kernel(
        _agg_body,
        out_type=jax.ShapeDtypeStruct((NC, N_PAD, F), jnp.float32),
        mesh=mesh,
        compiler_params=pltpu.CompilerParams(use_tc_tiling_on_sc=False),
        scratch_types=[
            pltpu.VMEM_SHARED((N_ACC, F), jnp.float32),
            pltpu.VMEM((2, SUP), jnp.int32),
            pltpu.VMEM((2, SUP), jnp.int32),
            pltpu.VMEM((2, SUP, F), jnp.float32),
            pltpu.SemaphoreType.DMA,
            pltpu.SemaphoreType.DMA,
            pltpu.SemaphoreType.DMA,
            pltpu.SemaphoreType.DMA,
            pltpu.SemaphoreType.DMA,
            pltpu.SemaphoreType.DMA,
            pltpu.SemaphoreType.DMA,
            pltpu.SemaphoreType.DMA,
        ],
    )(feat, src1, dst1, zrow)


def _prescale_body(degp, xb, dis_o, xp_o):
    deg = jnp.sum(degp[...], axis=0) + 1.0
    dis = lax.rsqrt(deg)
    eye = jnp.eye(128, dtype=jnp.float32)
    ones = jnp.ones((128, F), jnp.float32)
    pieces = []
    for q in range(R // 128):
        d = jnp.broadcast_to(dis[q:q + 1, :], (128, 128)) * eye
        pieces.append(jnp.dot(d, ones, preferred_element_type=jnp.float32))
    dis16 = jnp.concatenate(pieces, axis=0)
    dis_o[...] = dis16
    xp_o[...] = xb[...] * dis16


def _tc_prescale(deg_parts, x16):
    return pl.pallas_call(
        _prescale_body,
        grid=(TC_GRID,),
        in_specs=[
            pl.BlockSpec((NW, R // 128, 128), lambda i: (0, i, 0)),
            pl.BlockSpec((R, F), lambda i: (i, 0)),
        ],
        out_specs=[
            pl.BlockSpec((R, F), lambda i: (i, 0)),
            pl.BlockSpec((R, F), lambda i: (i, 0)),
        ],
        out_shape=[
            jax.ShapeDtypeStruct((N_PAD, F), jnp.float32),
            jax.ShapeDtypeStruct((N_PAD, F), jnp.float32),
        ],
    )(deg_parts, x16)


def _mid_body(agg, xp, dis, w1, b1, w2, gp_o):
    t = dis[...] * (agg[0] + agg[1] + xp[...])
    h1 = jnp.dot(t, w1[...], preferred_element_type=jnp.float32) + b1[...]
    h1 = jnp.maximum(h1, 0.0)
    g = jnp.dot(h1, w2[...], preferred_element_type=jnp.float32)
    gp_o[...] = dis[...] * g


def _tc_mid(agg1, xp, dis16, w1p, b1r, w2p):
    return pl.pallas_call(
        _mid_body,
        grid=(TC_GRID,),
        in_specs=[
            pl.BlockSpec((NC, R, F), lambda i: (0, i, 0)),
            pl.BlockSpec((R, F), lambda i: (i, 0)),
            pl.BlockSpec((R, F), lambda i: (i, 0)),
            pl.BlockSpec((F, 64), lambda i: (0, 0)),
            pl.BlockSpec((1, 64), lambda i: (0, 0)),
            pl.BlockSpec((64, F), lambda i: (0, 0)),
        ],
        out_specs=pl.BlockSpec((R, F), lambda i: (i, 0)),
        out_shape=jax.ShapeDtypeStruct((N_PAD, F), jnp.float32),
    )(agg1, xp, dis16, w1p, b1r, w2p)


def _final_body(agg, gp, dis, b2m, o):
    h2 = dis[...] * (agg[0] + agg[1] + gp[...]) + b2m[...]
    m = jnp.max(h2, axis=1, keepdims=True)
    e = jnp.exp(h2 - m)
    se = jnp.sum(e, axis=1, keepdims=True)
    o[...] = (h2 - m - jnp.log(se))[:, :10]


def _tc_final(agg2, gp, dis16, b2m):
    return pl.pallas_call(
        _final_body,
        grid=(N_NODES // RF,),
        in_specs=[
            pl.BlockSpec((NC, RF, F), lambda i: (0, i, 0)),
            pl.BlockSpec((RF, F), lambda i: (i, 0)),
            pl.BlockSpec((RF, F), lambda i: (i, 0)),
            pl.BlockSpec((1, F), lambda i: (0, 0)),
        ],
        out_specs=pl.BlockSpec((RF, 10), lambda i: (i, 0)),
        out_shape=jax.ShapeDtypeStruct((N_NODES, 10), jnp.float32),
    )(agg2, gp, dis16, b2m)


def kernel(x, edge_index, W1, b1, W2, b2):
    src = edge_index[0].astype(jnp.int32)
    dst = edge_index[1].astype(jnp.int32)

    x16 = jnp.pad(x, ((0, N_PAD - N_NODES), (0, F - x.shape[1])))
    w1p = jnp.pad(W1, ((0, F - W1.shape[0]), (0, 0)))
    w2p = jnp.pad(W2, ((0, 0), (0, F - W2.shape[1])))
    b1r = b1.reshape(1, 64)
    b2m = jnp.concatenate([b2, jnp.full((F - b2.shape[0],), -1e30, b2.dtype)])
    b2m = b2m.reshape(1, F)
    zrow = jnp.zeros((ROWS_PER_TILE, F), jnp.float32)
    zdeg = jnp.zeros((N_PAD,), jnp.float32)

    deg_flat = _deg_call(dst, zdeg)
    deg_parts = deg_flat.reshape(NW, N_PAD // 128, 128)
    dis16, xp = _tc_prescale(deg_parts, x16)
    agg1 = _agg_call(xp, src, dst, zrow)
    gp = _tc_mid(agg1, xp, dis16, w1p, b1r, w2p)
    agg2 = _agg_call(gp, src, dst, zrow)
    return _tc_final(agg2, gp, dis16, b2m)
